# baseline (speedup 1.0000x reference)
import math
import functools

import jax
import jax.numpy as jnp
from jax import lax
from jax.experimental import pallas as pl
from jax.experimental.pallas import tpu as pltpu
from jax.experimental.pallas import tpu_sc as plsc

VOCAB = 1000000
EMBED = 64
D_MODEL = 128
B = 4096
L = 200

NC = 2
NS = 16
NW = NC * NS

N = B * L
BLOCK = 12800
HALF = BLOCK // 2
NSLICE = 4
SBLK = 16
SROWS = SBLK * HALF
CHUNK = 2 * L
NCHUNK = HALF // CHUNK
SUBS = ((0, 0, 128), (128, 128, 72), (200, 0, 128), (328, 128, 72))
MMU = 1
MBLK = MMU * BLOCK


def _sc_gather_slice(x, table, s):
    mesh = plsc.VectorSubcoreMesh(core_axis_name="c", subcore_axis_name="s")

    @functools.partial(
        pl.kernel,
        out_type=jax.ShapeDtypeStruct((SROWS, 2 * EMBED), jnp.float32),
        mesh=mesh,
        scratch_types=[
            pltpu.VMEM((2 * NCHUNK, 256), jnp.int32),
            pltpu.VMEM((CHUNK, EMBED), jnp.float32),
            pltpu.VMEM((CHUNK, EMBED), jnp.float32),
            pltpu.SemaphoreType.DMA,
        ],
        compiler_params=pltpu.CompilerParams(use_tc_tiling_on_sc=False),
    )
    def k(idx_hbm, table_hbm, emb_hbm, idx_v, rows_v0, rows_v1, sem):
        wid = lax.axis_index("s") * NC + lax.axis_index("c")
        blk = wid // 2
        h = wid % 2
        pltpu.sync_copy(
            idx_hbm.at[pl.ds((s * SBLK + blk) * 64 + h * 32, 2 * NCHUNK)],
            idx_v,
        )

        def fire(t, buf):
            return [
                pltpu.async_copy(
                    table_hbm.at[idx_v.at[2 * t + do // L, pl.ds(co, n)]],
                    buf.at[pl.ds(do, n)],
                    sem,
                )
                for do, co, n in SUBS
            ]

        def flush(t, buf):
            pltpu.sync_copy(
                buf,
                emb_hbm.at[
                    pl.ds(blk * HALF + t * CHUNK, CHUNK),
                    pl.ds(h * EMBED, EMBED),
                ],
            )

        def pair_body(u, carry):
            descs = fire(2 * u, rows_v0)

            @pl.when(u > 0)
            def _():
                flush(2 * u - 1, rows_v1)

            for d in descs:
                d.wait()
            descs = fire(2 * u + 1, rows_v1)
            flush(2 * u, rows_v0)
            for d in descs:
                d.wait()
            return carry

        lax.fori_loop(0, NCHUNK // 2, pair_body, 0)
        flush(NCHUNK - 1, rows_v1)

    return k(x, table)


def _tc_matmul_slice(prev, emb_s, Wa, Wb, b2, s):
    def body(*refs):
        emb_ref, wa_ref, wb_ref, b_ref, out_ref = refs[-5:]
        for u in range(MMU):
            e = emb_ref[pl.ds(u * HALF, HALF), :]
            top = jnp.dot(e, wa_ref[...], preferred_element_type=jnp.float32)
            bot = jnp.dot(e, wb_ref[...], preferred_element_type=jnp.float32)
            r0 = u * BLOCK
            out_ref[pl.ds(r0, HALF), :] = top + b_ref[...]
            out_ref[pl.ds(r0 + HALF, HALF), :] = bot + b_ref[...]

    data_specs = [
        pl.BlockSpec((MMU * HALF, 2 * EMBED), lambda i: (i, 0)),
        pl.BlockSpec((2 * EMBED, D_MODEL), lambda i: (0, 0)),
        pl.BlockSpec((2 * EMBED, D_MODEL), lambda i: (0, 0)),
        pl.BlockSpec((1, D_MODEL), lambda i: (0, 0)),
    ]
    prev_args = () if prev is None else (prev,)
    prev_specs = [] if prev is None else [pl.BlockSpec(memory_space=pl.ANY)]
    return pl.pallas_call(
        body,
        grid=(SROWS // (MMU * HALF),),
        in_specs=prev_specs + data_specs,
        out_specs=pl.BlockSpec(
            (MBLK, D_MODEL), lambda i, s=s: (s * (SBLK // MMU) + i, 0)),
        out_shape=jax.ShapeDtypeStruct((N, D_MODEL), jnp.float32),
        input_output_aliases={} if prev is None else {0: 0},
    )(*prev_args, emb_s, Wa, Wb, b2)


def kernel(x, table, W, b):
    scale = math.sqrt(D_MODEL)
    t2 = jnp.pad(table, ((0, 0), (0, 2 * EMBED - table.shape[1])))
    t2 = t2.reshape(2 * VOCAB, EMBED)
    xp = jnp.pad(x.astype(jnp.int32) * 2, ((0, 0), (0, 256 - L)))
    Ws = W * scale
    zero = jnp.zeros_like(Ws)
    Wa = jnp.concatenate([Ws, zero], axis=0)
    Wb = jnp.concatenate([zero, Ws], axis=0)
    b2 = (b * scale).reshape(1, D_MODEL)

    embs = [_sc_gather_slice(xp, t2, s) for s in range(NSLICE)]
    out = None
    for s in range(NSLICE):
        out = _tc_matmul_slice(out, embs[s], Wa, Wb, b2, s)
    return out.reshape(B, L, D_MODEL)


# speedup vs baseline: 1.0050x; 1.0050x over previous
import math
import functools

import jax
import jax.numpy as jnp
from jax import lax
from jax.experimental import pallas as pl
from jax.experimental.pallas import tpu as pltpu
from jax.experimental.pallas import tpu_sc as plsc

VOCAB = 1000000
EMBED = 64
D_MODEL = 128
B = 4096
L = 200

NC = 2
NS = 16
NW = NC * NS

N = B * L
BLOCK = 12800
HALF = BLOCK // 2
NSLICE = 4
SBLK = 16
SROWS = SBLK * HALF
CHUNK = 2 * L
NCHUNK = HALF // CHUNK
SUBS = ((0, 0, 128), (128, 128, 72), (200, 0, 128), (328, 128, 72))
MMU = 2
MBLK = MMU * BLOCK


def _sc_gather_slice(x, table, s):
    mesh = plsc.VectorSubcoreMesh(core_axis_name="c", subcore_axis_name="s")

    @functools.partial(
        pl.kernel,
        out_type=jax.ShapeDtypeStruct((SROWS, 2 * EMBED), jnp.float32),
        mesh=mesh,
        scratch_types=[
            pltpu.VMEM((2 * NCHUNK, 256), jnp.int32),
            pltpu.VMEM((CHUNK, EMBED), jnp.float32),
            pltpu.VMEM((CHUNK, EMBED), jnp.float32),
            pltpu.SemaphoreType.DMA,
        ],
        compiler_params=pltpu.CompilerParams(use_tc_tiling_on_sc=False),
    )
    def k(idx_hbm, table_hbm, emb_hbm, idx_v, rows_v0, rows_v1, sem):
        wid = lax.axis_index("s") * NC + lax.axis_index("c")
        blk = wid // 2
        h = wid % 2
        pltpu.sync_copy(
            idx_hbm.at[pl.ds((s * SBLK + blk) * 64 + h * 32, 2 * NCHUNK)],
            idx_v,
        )

        def fire(t, buf):
            return [
                pltpu.async_copy(
                    table_hbm.at[idx_v.at[2 * t + do // L, pl.ds(co, n)]],
                    buf.at[pl.ds(do, n)],
                    sem,
                )
                for do, co, n in SUBS
            ]

        def flush(t, buf):
            pltpu.sync_copy(
                buf,
                emb_hbm.at[
                    pl.ds(blk * HALF + t * CHUNK, CHUNK),
                    pl.ds(h * EMBED, EMBED),
                ],
            )

        def pair_body(u, carry):
            descs = fire(2 * u, rows_v0)

            @pl.when(u > 0)
            def _():
                flush(2 * u - 1, rows_v1)

            for d in descs:
                d.wait()
            descs = fire(2 * u + 1, rows_v1)
            flush(2 * u, rows_v0)
            for d in descs:
                d.wait()
            return carry

        lax.fori_loop(0, NCHUNK // 2, pair_body, 0)
        flush(NCHUNK - 1, rows_v1)

    return k(x, table)


def _tc_matmul_slice(prev, emb_s, Wa, Wb, b2, s):
    def body(*refs):
        emb_ref, wa_ref, wb_ref, b_ref, out_ref = refs[-5:]
        for u in range(MMU):
            e = emb_ref[pl.ds(u * HALF, HALF), :]
            top = jnp.dot(e, wa_ref[...], preferred_element_type=jnp.float32)
            bot = jnp.dot(e, wb_ref[...], preferred_element_type=jnp.float32)
            r0 = u * BLOCK
            out_ref[pl.ds(r0, HALF), :] = top + b_ref[...]
            out_ref[pl.ds(r0 + HALF, HALF), :] = bot + b_ref[...]

    data_specs = [
        pl.BlockSpec((MMU * HALF, 2 * EMBED), lambda i: (i, 0)),
        pl.BlockSpec((2 * EMBED, D_MODEL), lambda i: (0, 0)),
        pl.BlockSpec((2 * EMBED, D_MODEL), lambda i: (0, 0)),
        pl.BlockSpec((1, D_MODEL), lambda i: (0, 0)),
    ]
    prev_args = () if prev is None else (prev,)
    prev_specs = [] if prev is None else [pl.BlockSpec(memory_space=pl.ANY)]
    return pl.pallas_call(
        body,
        grid=(SROWS // (MMU * HALF),),
        in_specs=prev_specs + data_specs,
        out_specs=pl.BlockSpec(
            (MBLK, D_MODEL), lambda i, s=s: (s * (SBLK // MMU) + i, 0)),
        out_shape=jax.ShapeDtypeStruct((N, D_MODEL), jnp.float32),
        input_output_aliases={} if prev is None else {0: 0},
    )(*prev_args, emb_s, Wa, Wb, b2)


def kernel(x, table, W, b):
    scale = math.sqrt(D_MODEL)
    t2 = jnp.pad(table, ((0, 0), (0, 2 * EMBED - table.shape[1])))
    t2 = t2.reshape(2 * VOCAB, EMBED)
    xp = jnp.pad(x.astype(jnp.int32) * 2, ((0, 0), (0, 256 - L)))
    Ws = W * scale
    zero = jnp.zeros_like(Ws)
    Wa = jnp.concatenate([Ws, zero], axis=0)
    Wb = jnp.concatenate([zero, Ws], axis=0)
    b2 = (b * scale).reshape(1, D_MODEL)

    embs = [_sc_gather_slice(xp, t2, s) for s in range(NSLICE)]
    out = None
    for s in range(NSLICE):
        out = _tc_matmul_slice(out, embs[s], Wa, Wb, b2, s)
    return out.reshape(B, L, D_MODEL)


# speedup vs baseline: 1.0099x; 1.0049x over previous
import math
import functools

import jax
import jax.numpy as jnp
from jax import lax
from jax.experimental import pallas as pl
from jax.experimental.pallas import tpu as pltpu
from jax.experimental.pallas import tpu_sc as plsc

VOCAB = 1000000
EMBED = 64
D_MODEL = 128
B = 4096
L = 200

NC = 2
NS = 16
NW = NC * NS

N = B * L
BLOCK = 12800
HALF = BLOCK // 2
NSLICE = 4
SBLK = 16
SROWS = SBLK * HALF
CHUNK = 4 * L
NCHUNK = HALF // CHUNK
SUBS = tuple(
    (r * L + c0, c0, n)
    for r in range(CHUNK // L)
    for c0, n in ((0, 128), (128, 72))
)
MMU = 2
MBLK = MMU * BLOCK


def _sc_gather_slice(x, table, s):
    mesh = plsc.VectorSubcoreMesh(core_axis_name="c", subcore_axis_name="s")

    @functools.partial(
        pl.kernel,
        out_type=jax.ShapeDtypeStruct((SROWS, 2 * EMBED), jnp.float32),
        mesh=mesh,
        scratch_types=[
            pltpu.VMEM((HALF // L, 256), jnp.int32),
            pltpu.VMEM((CHUNK, EMBED), jnp.float32),
            pltpu.VMEM((CHUNK, EMBED), jnp.float32),
            pltpu.SemaphoreType.DMA,
        ],
        compiler_params=pltpu.CompilerParams(use_tc_tiling_on_sc=False),
    )
    def k(idx_hbm, table_hbm, emb_hbm, idx_v, rows_v0, rows_v1, sem):
        wid = lax.axis_index("s") * NC + lax.axis_index("c")
        blk = wid // 2
        h = wid % 2
        pltpu.sync_copy(
            idx_hbm.at[pl.ds((s * SBLK + blk) * 64 + h * 32, HALF // L)],
            idx_v,
        )

        def fire(t, buf):
            return [
                pltpu.async_copy(
                    table_hbm.at[
                        idx_v.at[(CHUNK // L) * t + do // L, pl.ds(co, n)]],
                    buf.at[pl.ds(do, n)],
                    sem,
                )
                for do, co, n in SUBS
            ]

        def flush(t, buf):
            pltpu.sync_copy(
                buf,
                emb_hbm.at[
                    pl.ds(blk * HALF + t * CHUNK, CHUNK),
                    pl.ds(h * EMBED, EMBED),
                ],
            )

        def pair_body(u, carry):
            descs = fire(2 * u, rows_v0)

            @pl.when(u > 0)
            def _():
                flush(2 * u - 1, rows_v1)

            for d in descs:
                d.wait()
            descs = fire(2 * u + 1, rows_v1)
            flush(2 * u, rows_v0)
            for d in descs:
                d.wait()
            return carry

        lax.fori_loop(0, NCHUNK // 2, pair_body, 0)
        flush(NCHUNK - 1, rows_v1)

    return k(x, table)


def _tc_matmul_slice(prev, emb_s, Wa, Wb, b2, s):
    def body(*refs):
        emb_ref, wa_ref, wb_ref, b_ref, out_ref = refs[-5:]
        for u in range(MMU):
            e = emb_ref[pl.ds(u * HALF, HALF), :]
            top = jnp.dot(e, wa_ref[...], preferred_element_type=jnp.float32)
            bot = jnp.dot(e, wb_ref[...], preferred_element_type=jnp.float32)
            r0 = u * BLOCK
            out_ref[pl.ds(r0, HALF), :] = top + b_ref[...]
            out_ref[pl.ds(r0 + HALF, HALF), :] = bot + b_ref[...]

    data_specs = [
        pl.BlockSpec((MMU * HALF, 2 * EMBED), lambda i: (i, 0)),
        pl.BlockSpec((2 * EMBED, D_MODEL), lambda i: (0, 0)),
        pl.BlockSpec((2 * EMBED, D_MODEL), lambda i: (0, 0)),
        pl.BlockSpec((1, D_MODEL), lambda i: (0, 0)),
    ]
    prev_args = () if prev is None else (prev,)
    prev_specs = [] if prev is None else [pl.BlockSpec(memory_space=pl.ANY)]
    return pl.pallas_call(
        body,
        grid=(SROWS // (MMU * HALF),),
        in_specs=prev_specs + data_specs,
        out_specs=pl.BlockSpec(
            (MBLK, D_MODEL), lambda i, s=s: (s * (SBLK // MMU) + i, 0)),
        out_shape=jax.ShapeDtypeStruct((N, D_MODEL), jnp.float32),
        input_output_aliases={} if prev is None else {0: 0},
    )(*prev_args, emb_s, Wa, Wb, b2)


def kernel(x, table, W, b):
    scale = math.sqrt(D_MODEL)
    t2 = jnp.pad(table, ((0, 0), (0, 2 * EMBED - table.shape[1])))
    t2 = t2.reshape(2 * VOCAB, EMBED)
    xp = jnp.pad(x.astype(jnp.int32) * 2, ((0, 0), (0, 256 - L)))
    Ws = W * scale
    zero = jnp.zeros_like(Ws)
    Wa = jnp.concatenate([Ws, zero], axis=0)
    Wb = jnp.concatenate([zero, Ws], axis=0)
    b2 = (b * scale).reshape(1, D_MODEL)

    embs = [_sc_gather_slice(xp, t2, s) for s in range(NSLICE)]
    out = None
    for s in range(NSLICE):
        out = _tc_matmul_slice(out, embs[s], Wa, Wb, b2, s)
    return out.reshape(B, L, D_MODEL)
